# baseline (speedup 1.0000x reference)
import functools

import jax
import jax.numpy as jnp
from jax import lax
from jax.experimental import pallas as pl
from jax.experimental.pallas import tpu as pltpu
from jax.experimental.pallas import tpu_sc as plsc

_NC = 2
_NS = 16
_NW = _NC * _NS
_CHUNK = 128
_IDXW = 128
_HP = 128


def _proj_body(twin, x_ref, w_ref, b_ref, *o_refs):
    v = jnp.maximum(
        jnp.dot(x_ref[...], w_ref[...], preferred_element_type=jnp.float32)
        + b_ref[...], 0.0)
    for o in o_refs:
        o[...] = v


def _proj(x, w, b2d, block_rows, twin=False):
    rows, k = x.shape
    h = w.shape[1]
    n_out = 2 if twin else 1
    out_spec = pl.BlockSpec((block_rows, h), lambda i: (i, 0))
    out_shape = jax.ShapeDtypeStruct((rows, h), jnp.float32)
    return pl.pallas_call(
        functools.partial(_proj_body, twin),
        grid=(rows // block_rows,),
        in_specs=[
            pl.BlockSpec((block_rows, k), lambda i: (i, 0)),
            pl.BlockSpec((k, h), lambda i: (0, 0)),
            pl.BlockSpec((1, h), lambda i: (0, 0)),
        ],
        out_specs=[out_spec] * n_out if twin else out_spec,
        out_shape=[out_shape] * n_out if twin else out_shape,
    )(x, w, b2d)


def _layer_body(nparts, with_mean, twin, *refs):
    h_ref = refs[0]
    part_refs = refs[1:1 + nparts]
    ws_ref, wm_ref, b_ref = refs[1 + nparts:4 + nparts]
    o_refs = refs[4 + nparts:4 + nparts + (2 if twin else 1)]
    agg = part_refs[0][...]
    for r in part_refs[1:]:
        agg = agg + r[...]
    o = jnp.maximum(
        jnp.dot(h_ref[...], ws_ref[...], preferred_element_type=jnp.float32)
        + jnp.dot(agg, wm_ref[...], preferred_element_type=jnp.float32)
        + b_ref[...], 0.0)
    for o_ref in o_refs:
        o_ref[...] = o
    if with_mean:
        ms_ref = refs[-1]

        @pl.when(pl.program_id(0) == 0)
        def _():
            ms_ref[...] = jnp.zeros_like(ms_ref)

        ms_ref[...] += jnp.sum(o, axis=0, keepdims=True)


def _layer(h, parts, w_self, w_msg, b2d, block_rows, with_mean, twin=False):
    n, hd = h.shape
    nparts = len(parts)
    grid = n // block_rows
    part_spec = pl.BlockSpec((block_rows, hd), lambda i: (i, 0))
    in_specs = (
        [pl.BlockSpec((block_rows, hd), lambda i: (i, 0))]
        + [part_spec] * nparts
        + [pl.BlockSpec((hd, hd), lambda i: (0, 0))] * 2
        + [pl.BlockSpec((1, hd), lambda i: (0, 0))]
    )
    o_spec = pl.BlockSpec((block_rows, hd), lambda i: (i, 0))
    o_shape = jax.ShapeDtypeStruct((n, hd), jnp.float32)
    out_specs = [o_spec] * (2 if twin else 1)
    out_shape = [o_shape] * (2 if twin else 1)
    if with_mean:
        out_specs.append(pl.BlockSpec((1, hd), lambda i: (0, 0)))
        out_shape.append(jax.ShapeDtypeStruct((1, hd), jnp.float32))
    res = pl.pallas_call(
        functools.partial(_layer_body, nparts, with_mean, twin),
        grid=(grid,),
        in_specs=in_specs,
        out_specs=out_specs,
        out_shape=out_shape,
    )(h, *parts, w_self, w_msg, b2d)
    return res


def _head_body(n_nodes, o0_ref, o1_ref, ws1a_ref, ws1b_ref, bs1_ref, ws2t_ref,
               bs2_ref, ms_ref, ts_ref, wk1h_ref, wk1t_ref, bk1_ref, wk2_ref,
               bk2_ref, wk3_ref, bk3_ref, sc_ref, kc_ref, ki_ref):
    z = jnp.maximum(
        jnp.dot(o0_ref[...], ws1a_ref[...], preferred_element_type=jnp.float32)
        + jnp.dot(o1_ref[...], ws1b_ref[...], preferred_element_type=jnp.float32)
        + bs1_ref[...], 0.0)
    sc_ref[...] = lax.dot_general(
        ws2t_ref[...], z, (((1,), (1,)), ((), ())),
        preferred_element_type=jnp.float32) + bs2_ref[...]
    ge = ms_ref[...] * (1.0 / n_nodes)
    h1 = jnp.maximum(
        jnp.dot(ge, wk1h_ref[...], preferred_element_type=jnp.float32)
        + jnp.dot(ts_ref[...], wk1t_ref[...], preferred_element_type=jnp.float32)
        + bk1_ref[...], 0.0)
    h2 = jnp.maximum(
        jnp.dot(h1, wk2_ref[...], preferred_element_type=jnp.float32)
        + bk2_ref[...], 0.0)
    raw = jnp.dot(h2, wk3_ref[...], preferred_element_type=jnp.float32) + bk3_ref[...]
    kc = 1.0 + 49.0 / (1.0 + jnp.exp(-raw))
    kc_ref[...] = kc
    ki_ref[...] = jnp.clip(jnp.round(kc), 1.0, 50.0).astype(jnp.int32)


def _acc_copies(zr):
    pieces, off = [], 0
    while off < zr:
        sz = min(_CHUNK, zr - off)
        pieces.append((off, sz))
        off += sz
    return pieces


def _build_segsum(n_acc, ep, gather, split=None):
    pw = ep // _NW
    ch = pw // _CHUNK
    if split is None:
        split = (ch, ch)
    ir = max(split)
    zr = n_acc // _NS
    mesh = plsc.VectorSubcoreMesh(core_axis_name="c", subcore_axis_name="s")
    out_type = [jax.ShapeDtypeStruct((n_acc, _HP), jnp.float32)] * _NC
    scratch = [
        pltpu.VMEM((2, _IDXW), jnp.int32),
        pltpu.VMEM((_CHUNK, _HP), jnp.float32),
        pltpu.VMEM((_CHUNK, _HP), jnp.float32),
        pltpu.VMEM_SHARED((n_acc, _HP), jnp.float32),
        pltpu.SemaphoreType.DMA,
        pltpu.SemaphoreType.DMA,
    ]
    if gather:
        scratch.insert(0, pltpu.VMEM((ir, _IDXW), jnp.int32))

    def body(*refs):
        if gather:
            tab0_hbm, tab1_hbm, src_hbm, dst_hbm = refs[:4]
            outs = refs[4:4 + _NC]
            src_a, dstp, buf0, buf1, acc, sem0, sem1 = refs[4 + _NC:]
        else:
            tab0_hbm, dst_hbm = refs[:2]
            outs = refs[2:2 + _NC]
            dstp, buf0, buf1, acc, sem0, sem1 = refs[2 + _NC:]
            src_a = None
        cid = lax.axis_index("c")
        sid = lax.axis_index("s")

        def zrow(i, carry):
            for j in range(_HP // 16):
                buf0[i, pl.ds(j * 16, 16)] = jnp.zeros((16,), jnp.float32)
            return carry

        lax.fori_loop(0, min(zr, _CHUNK), zrow, 0)
        for off, sz in _acc_copies(zr):
            pltpu.sync_copy(buf0.at[pl.ds(0, sz)],
                            acc.at[pl.ds(sid * zr + off, sz)])
        plsc.subcore_barrier()

        def run_core(tab_hbm, row0, nch):
            if gather:
                pltpu.sync_copy(src_hbm.at[pl.ds(row0, nch)],
                                src_a.at[pl.ds(0, nch)])

            def issue(t, buf, sem):
                if gather:
                    return pltpu.async_copy(tab_hbm.at[src_a.at[t]], buf, sem)
                return pltpu.async_copy(
                    tab_hbm.at[pl.ds((row0 + t) * _IDXW, _CHUNK)], buf, sem)

            def pair(i, carry):
                t0 = 2 * i
                d0 = issue(t0, buf0, sem0)
                d1 = issue(t0 + 1, buf1, sem1)
                pltpu.sync_copy(dst_hbm.at[pl.ds(row0 + t0, 2)], dstp)
                d0.wait()
                pltpu.sync_copy(buf0, acc.at[dstp.at[0]], add=True)
                d1.wait()
                pltpu.sync_copy(buf1, acc.at[dstp.at[1]], add=True)
                return carry

            lax.fori_loop(0, nch // 2, pair, 0)

        @pl.when(cid == 0)
        def _():
            run_core(tab0_hbm, sid * split[0], split[0])

        @pl.when(cid == 1)
        def _():
            run_core(tab1_hbm if gather else tab0_hbm,
                     _NS * split[0] + sid * split[1], split[1])

        plsc.subcore_barrier()

        for off, sz in _acc_copies(zr):
            pltpu.sync_copy(acc.at[pl.ds(sid * zr + off, sz)],
                            buf0.at[pl.ds(0, sz)])
            for c in range(_NC):
                @pl.when(cid == c)
                def _():
                    pltpu.sync_copy(buf0.at[pl.ds(0, sz)],
                                    outs[c].at[pl.ds(sid * zr + off, sz)])

    return pl.kernel(body, out_type=out_type, mesh=mesh,
                     scratch_types=scratch)


def _build_odgather(nod):
    pw = nod // _NW
    mesh = plsc.VectorSubcoreMesh(core_axis_name="c", subcore_axis_name="s")

    @functools.partial(
        pl.kernel,
        out_type=[jax.ShapeDtypeStruct((nod, _HP), jnp.float32),
                  jax.ShapeDtypeStruct((nod, _HP), jnp.float32)],
        mesh=mesh,
        scratch_types=[
            pltpu.VMEM((1, pw), jnp.int32),
            pltpu.VMEM((pw, _HP), jnp.float32),
            pltpu.SemaphoreType.DMA,
        ],
    )
    def body(h_hbm, od0_hbm, od1_hbm, o0, o1, idx_v, rows_v, sem):
        cid = lax.axis_index("c")
        sid = lax.axis_index("s")
        wid = sid * _NC + cid
        for odh, out in ((od0_hbm, o0), (od1_hbm, o1)):
            pltpu.sync_copy(odh.at[pl.ds(wid, 1)], idx_v)
            pltpu.async_copy(h_hbm.at[idx_v.at[0]], rows_v, sem).wait()
            pltpu.sync_copy(rows_v, out.at[pl.ds(wid * pw, pw)])

    return body


def _pad_w(w):
    r, c = w.shape
    return jnp.pad(w, ((0, _HP - r if r == 64 else 0),
                       (0, _HP - c if c == 64 else 0)))


def kernel(node_features, edge_features, traffic_stats, edge_index, od_pairs,
           W_node, b_node, W_edge, b_edge,
           W_self0, W_msg0, b0, W_self1, W_msg1, b1,
           W_s1, b_s1, W_s2, b_s2,
           W_k1, b_k1, W_k2, b_k2, W_k3, b_k3):
    n, _ = node_features.shape
    e_cnt, _ = edge_features.shape
    hd = W_node.shape[1]
    nod = od_pairs.shape[0]

    pw = -(-e_cnt // (_NW * 2 * _CHUNK)) * (2 * _CHUNK)
    ep = pw * _NW
    n_acc = (8 * _NS) * (-(-(n + 1) // (8 * _NS)))

    src = edge_index[0]
    dst = edge_index[1]
    pad = ep - e_cnt
    src2d = jnp.concatenate(
        [src, jnp.zeros((pad,), jnp.int32)]).reshape(ep // _IDXW, _IDXW)
    dst2d = jnp.concatenate(
        [dst, jnp.full((pad,), n, jnp.int32)]).reshape(ep // _IDXW, _IDXW)
    ef_p = jnp.pad(edge_features, ((0, pad), (0, 0)))

    wn_p = jnp.pad(W_node, ((0, 0), (0, _HP - hd)))
    we_p = jnp.pad(W_edge, ((0, 0), (0, _HP - hd)))
    bn_p = jnp.pad(b_node, (0, _HP - hd)).reshape(1, _HP)
    be_p = jnp.pad(b_edge, (0, _HP - hd)).reshape(1, _HP)
    ws0_p, wm0_p = _pad_w(W_self0), _pad_w(W_msg0)
    ws1_p, wm1_p = _pad_w(W_self1), _pad_w(W_msg1)
    b0_p = jnp.pad(b0, (0, _HP - hd)).reshape(1, _HP)
    b1_p = jnp.pad(b1, (0, _HP - hd)).reshape(1, _HP)

    h0a, h0b = _proj(node_features, wn_p, bn_p, 1000, twin=True)
    e = _proj(ef_p, we_p, be_p, 2048)

    c_tot = ep // (_NS * _CHUNK)
    c0 = (c_tot * 2 // 3) // 8 * 8
    split = (c_tot - c0, c0)

    se_parts = _build_segsum(n_acc, ep, gather=False)(e, dst2d)
    g0_parts = _build_segsum(n_acc, ep, gather=True,
                             split=split)(h0a, h0b, src2d, dst2d)

    h1a, h1b = _layer(h0a, list(g0_parts) + list(se_parts), ws0_p, wm0_p,
                      b0_p, 1000, with_mean=False, twin=True)

    g1_parts = _build_segsum(n_acc, ep, gather=True,
                             split=split)(h1a, h1b, src2d, dst2d)
    h2, msum = _layer(h1a, list(g1_parts) + list(se_parts), ws1_p, wm1_p,
                      b1_p, 1000, with_mean=True)

    od0 = od_pairs[:, 0].reshape(_NW, nod // _NW)
    od1 = od_pairs[:, 1].reshape(_NW, nod // _NW)
    o0, o1 = _build_odgather(nod)(h2, od0, od1)

    ws1a_p = jnp.pad(W_s1[:hd], ((0, _HP - hd), (0, 0)))
    ws1b_p = jnp.pad(W_s1[hd:], ((0, _HP - hd), (0, 0)))
    wk1h_p = jnp.pad(W_k1[:hd], ((0, _HP - hd), (0, 0)))
    scores_t, kc, ki = pl.pallas_call(
        functools.partial(_head_body, n),
        out_shape=[
            jax.ShapeDtypeStruct((1, nod), jnp.float32),
            jax.ShapeDtypeStruct((1, 1), jnp.float32),
            jax.ShapeDtypeStruct((1, 1), jnp.int32),
        ],
    )(o0, o1, ws1a_p, ws1b_p, b_s1.reshape(1, hd),
      W_s2.reshape(1, hd), b_s2.reshape(1, 1), msum,
      traffic_stats.reshape(1, 4), wk1h_p, W_k1[hd:],
      b_k1.reshape(1, 32), W_k2, b_k2.reshape(1, 16), W_k3,
      b_k3.reshape(1, 1))

    return (scores_t.reshape(nod), kc.reshape(()), ki.reshape(()))


# speedup vs baseline: 1.1019x; 1.1019x over previous
import functools

import jax
import jax.numpy as jnp
from jax import lax
from jax.experimental import pallas as pl
from jax.experimental.pallas import tpu as pltpu
from jax.experimental.pallas import tpu_sc as plsc

_NC = 2
_NS = 16
_NW = _NC * _NS
_CHUNK = 128
_IDXW = 128
_HP = 128


def _proj_body(twin, x_ref, w_ref, b_ref, *o_refs):
    v = jnp.maximum(
        jnp.dot(x_ref[...], w_ref[...], preferred_element_type=jnp.float32)
        + b_ref[...], 0.0)
    for o in o_refs:
        o[...] = v


def _proj(x, w, b2d, block_rows, twin=False):
    rows, k = x.shape
    h = w.shape[1]
    n_out = 2 if twin else 1
    out_spec = pl.BlockSpec((block_rows, h), lambda i: (i, 0))
    out_shape = jax.ShapeDtypeStruct((rows, h), jnp.float32)
    return pl.pallas_call(
        functools.partial(_proj_body, twin),
        grid=(rows // block_rows,),
        in_specs=[
            pl.BlockSpec((block_rows, k), lambda i: (i, 0)),
            pl.BlockSpec((k, h), lambda i: (0, 0)),
            pl.BlockSpec((1, h), lambda i: (0, 0)),
        ],
        out_specs=[out_spec] * n_out if twin else out_spec,
        out_shape=[out_shape] * n_out if twin else out_shape,
    )(x, w, b2d)


def _layer_body(nparts, with_mean, twin, *refs):
    h_ref = refs[0]
    part_refs = refs[1:1 + nparts]
    ws_ref, wm_ref, b_ref = refs[1 + nparts:4 + nparts]
    o_refs = refs[4 + nparts:4 + nparts + (2 if twin else 1)]
    agg = part_refs[0][...]
    for r in part_refs[1:]:
        agg = agg + r[...]
    o = jnp.maximum(
        jnp.dot(h_ref[...], ws_ref[...], preferred_element_type=jnp.float32)
        + jnp.dot(agg, wm_ref[...], preferred_element_type=jnp.float32)
        + b_ref[...], 0.0)
    for o_ref in o_refs:
        o_ref[...] = o
    if with_mean:
        ms_ref = refs[-1]

        @pl.when(pl.program_id(0) == 0)
        def _():
            ms_ref[...] = jnp.zeros_like(ms_ref)

        ms_ref[...] += jnp.sum(o, axis=0, keepdims=True)


def _layer(h, parts, w_self, w_msg, b2d, block_rows, with_mean, twin=False):
    n, hd = h.shape
    nparts = len(parts)
    grid = n // block_rows
    part_spec = pl.BlockSpec((block_rows, hd), lambda i: (i, 0))
    in_specs = (
        [pl.BlockSpec((block_rows, hd), lambda i: (i, 0))]
        + [part_spec] * nparts
        + [pl.BlockSpec((hd, hd), lambda i: (0, 0))] * 2
        + [pl.BlockSpec((1, hd), lambda i: (0, 0))]
    )
    o_spec = pl.BlockSpec((block_rows, hd), lambda i: (i, 0))
    o_shape = jax.ShapeDtypeStruct((n, hd), jnp.float32)
    out_specs = [o_spec] * (2 if twin else 1)
    out_shape = [o_shape] * (2 if twin else 1)
    if with_mean:
        out_specs.append(pl.BlockSpec((1, hd), lambda i: (0, 0)))
        out_shape.append(jax.ShapeDtypeStruct((1, hd), jnp.float32))
    res = pl.pallas_call(
        functools.partial(_layer_body, nparts, with_mean, twin),
        grid=(grid,),
        in_specs=in_specs,
        out_specs=out_specs,
        out_shape=out_shape,
    )(h, *parts, w_self, w_msg, b2d)
    return res


def _head_body(n_nodes, o0_ref, o1_ref, ws1a_ref, ws1b_ref, bs1_ref, ws2t_ref,
               bs2_ref, ms_ref, ts_ref, wk1h_ref, wk1t_ref, bk1_ref, wk2_ref,
               bk2_ref, wk3_ref, bk3_ref, sc_ref, kc_ref, ki_ref):
    z = jnp.maximum(
        jnp.dot(o0_ref[...], ws1a_ref[...], preferred_element_type=jnp.float32)
        + jnp.dot(o1_ref[...], ws1b_ref[...], preferred_element_type=jnp.float32)
        + bs1_ref[...], 0.0)
    sc_ref[...] = lax.dot_general(
        ws2t_ref[...], z, (((1,), (1,)), ((), ())),
        preferred_element_type=jnp.float32) + bs2_ref[...]
    ge = ms_ref[...] * (1.0 / n_nodes)
    h1 = jnp.maximum(
        jnp.dot(ge, wk1h_ref[...], preferred_element_type=jnp.float32)
        + jnp.dot(ts_ref[...], wk1t_ref[...], preferred_element_type=jnp.float32)
        + bk1_ref[...], 0.0)
    h2 = jnp.maximum(
        jnp.dot(h1, wk2_ref[...], preferred_element_type=jnp.float32)
        + bk2_ref[...], 0.0)
    raw = jnp.dot(h2, wk3_ref[...], preferred_element_type=jnp.float32) + bk3_ref[...]
    kc = 1.0 + 49.0 / (1.0 + jnp.exp(-raw))
    kc_ref[...] = kc
    ki_ref[...] = jnp.clip(jnp.round(kc), 1.0, 50.0).astype(jnp.int32)


def _acc_copies(zr):
    pieces, off = [], 0
    while off < zr:
        sz = min(_CHUNK, zr - off)
        pieces.append((off, sz))
        off += sz
    return pieces


def _build_segsum(n_acc, ep, gather, split=None):
    pw = ep // _NW
    ch = pw // _CHUNK
    if split is None:
        split = (ch, ch)
    ir = max(split)
    zr = n_acc // _NS
    mesh = plsc.VectorSubcoreMesh(core_axis_name="c", subcore_axis_name="s")
    out_type = [jax.ShapeDtypeStruct((n_acc, _HP), jnp.float32)] * _NC
    scratch = [
        pltpu.VMEM((2, _IDXW), jnp.int32),
        pltpu.VMEM((_CHUNK, _HP), jnp.float32),
        pltpu.VMEM((_CHUNK, _HP), jnp.float32),
        pltpu.VMEM_SHARED((n_acc, _HP), jnp.float32),
        pltpu.SemaphoreType.DMA,
        pltpu.SemaphoreType.DMA,
    ]
    if gather:
        scratch.insert(0, pltpu.VMEM((ir, _IDXW), jnp.int32))

    def body(*refs):
        if gather:
            tab0_hbm, tab1_hbm, src_hbm, dst_hbm = refs[:4]
            outs = refs[4:4 + _NC]
            src_a, dstp, buf0, buf1, acc, sem0, sem1 = refs[4 + _NC:]
        else:
            tab0_hbm, dst_hbm = refs[:2]
            outs = refs[2:2 + _NC]
            dstp, buf0, buf1, acc, sem0, sem1 = refs[2 + _NC:]
            src_a = None
        cid = lax.axis_index("c")
        sid = lax.axis_index("s")

        def zrow(i, carry):
            for j in range(_HP // 16):
                buf0[i, pl.ds(j * 16, 16)] = jnp.zeros((16,), jnp.float32)
            return carry

        lax.fori_loop(0, min(zr, _CHUNK), zrow, 0)
        for off, sz in _acc_copies(zr):
            pltpu.sync_copy(buf0.at[pl.ds(0, sz)],
                            acc.at[pl.ds(sid * zr + off, sz)])
        plsc.subcore_barrier()

        def run_core(tab_hbm, row0, nch):
            if gather:
                pltpu.sync_copy(src_hbm.at[pl.ds(row0, nch)],
                                src_a.at[pl.ds(0, nch)])

            def issue(t, buf, sem):
                if gather:
                    return pltpu.async_copy(tab_hbm.at[src_a.at[t]], buf, sem)
                return pltpu.async_copy(
                    tab_hbm.at[pl.ds((row0 + t) * _IDXW, _CHUNK)], buf, sem)

            def pair(i, carry):
                t0 = 2 * i
                d0 = issue(t0, buf0, sem0)
                d1 = issue(t0 + 1, buf1, sem1)
                pltpu.sync_copy(dst_hbm.at[pl.ds(row0 + t0, 2)], dstp)
                d0.wait()
                pltpu.sync_copy(buf0, acc.at[dstp.at[0]], add=True)
                d1.wait()
                pltpu.sync_copy(buf1, acc.at[dstp.at[1]], add=True)
                return carry

            lax.fori_loop(0, nch // 2, pair, 0)

        @pl.when(cid == 0)
        def _():
            run_core(tab0_hbm, sid * split[0], split[0])

        @pl.when(cid == 1)
        def _():
            run_core(tab1_hbm if gather else tab0_hbm,
                     _NS * split[0] + sid * split[1], split[1])

        plsc.subcore_barrier()

        for off, sz in _acc_copies(zr):
            pltpu.sync_copy(acc.at[pl.ds(sid * zr + off, sz)],
                            buf0.at[pl.ds(0, sz)])
            for c in range(_NC):
                @pl.when(cid == c)
                def _():
                    pltpu.sync_copy(buf0.at[pl.ds(0, sz)],
                                    outs[c].at[pl.ds(sid * zr + off, sz)])

    return pl.kernel(body, out_type=out_type, mesh=mesh,
                     scratch_types=scratch)


def _build_odgather(nod):
    pw = nod // _NW
    mesh = plsc.VectorSubcoreMesh(core_axis_name="c", subcore_axis_name="s")

    @functools.partial(
        pl.kernel,
        out_type=[jax.ShapeDtypeStruct((nod, _HP), jnp.float32),
                  jax.ShapeDtypeStruct((nod, _HP), jnp.float32)],
        mesh=mesh,
        scratch_types=[
            pltpu.VMEM((1, pw), jnp.int32),
            pltpu.VMEM((pw, _HP), jnp.float32),
            pltpu.SemaphoreType.DMA,
        ],
    )
    def body(h_hbm, od0_hbm, od1_hbm, o0, o1, idx_v, rows_v, sem):
        cid = lax.axis_index("c")
        sid = lax.axis_index("s")
        wid = sid * _NC + cid
        for odh, out in ((od0_hbm, o0), (od1_hbm, o1)):
            pltpu.sync_copy(odh.at[pl.ds(wid, 1)], idx_v)
            pltpu.async_copy(h_hbm.at[idx_v.at[0]], rows_v, sem).wait()
            pltpu.sync_copy(rows_v, out.at[pl.ds(wid * pw, pw)])

    return body


def _pad_w(w):
    r, c = w.shape
    return jnp.pad(w, ((0, _HP - r if r == 64 else 0),
                       (0, _HP - c if c == 64 else 0)))


def kernel(node_features, edge_features, traffic_stats, edge_index, od_pairs,
           W_node, b_node, W_edge, b_edge,
           W_self0, W_msg0, b0, W_self1, W_msg1, b1,
           W_s1, b_s1, W_s2, b_s2,
           W_k1, b_k1, W_k2, b_k2, W_k3, b_k3):
    n, _ = node_features.shape
    e_cnt, _ = edge_features.shape
    hd = W_node.shape[1]
    nod = od_pairs.shape[0]

    pw = -(-e_cnt // (_NW * 2 * _CHUNK)) * (2 * _CHUNK)
    ep = pw * _NW
    n_acc = (8 * _NS) * (-(-(n + 1) // (8 * _NS)))

    src = edge_index[0]
    dst = edge_index[1]
    pad = ep - e_cnt
    src2d = jnp.concatenate(
        [src, jnp.zeros((pad,), jnp.int32)]).reshape(ep // _IDXW, _IDXW)
    dst2d = jnp.concatenate(
        [dst, jnp.full((pad,), n, jnp.int32)]).reshape(ep // _IDXW, _IDXW)
    ef_p = jnp.pad(edge_features, ((0, pad), (0, 0)))

    wn_p = jnp.pad(W_node, ((0, 0), (0, _HP - hd)))
    we_p = jnp.pad(W_edge, ((0, 0), (0, _HP - hd)))
    bn_p = jnp.pad(b_node, (0, _HP - hd)).reshape(1, _HP)
    be_p = jnp.pad(b_edge, (0, _HP - hd)).reshape(1, _HP)
    ws0_p, wm0_p = _pad_w(W_self0), _pad_w(W_msg0)
    ws1_p, wm1_p = _pad_w(W_self1), _pad_w(W_msg1)
    b0_p = jnp.pad(b0, (0, _HP - hd)).reshape(1, _HP)
    b1_p = jnp.pad(b1, (0, _HP - hd)).reshape(1, _HP)

    h0a, h0b = _proj(node_features, wn_p, bn_p, 1000, twin=True)
    e = _proj(ef_p, we_p, be_p, 2048)

    c_tot = ep // (_NS * _CHUNK)
    c0 = (c_tot * 3 // 4) // 8 * 8
    split = (c0, c_tot - c0)

    se_parts = _build_segsum(n_acc, ep, gather=False)(e, dst2d)
    g0_parts = _build_segsum(n_acc, ep, gather=True,
                             split=split)(h0a, h0b, src2d, dst2d)

    h1a, h1b = _layer(h0a, list(g0_parts) + list(se_parts), ws0_p, wm0_p,
                      b0_p, 1000, with_mean=False, twin=True)

    g1_parts = _build_segsum(n_acc, ep, gather=True,
                             split=split)(h1a, h1b, src2d, dst2d)
    h2, msum = _layer(h1a, list(g1_parts) + list(se_parts), ws1_p, wm1_p,
                      b1_p, 1000, with_mean=True)

    od0 = od_pairs[:, 0].reshape(_NW, nod // _NW)
    od1 = od_pairs[:, 1].reshape(_NW, nod // _NW)
    o0, o1 = _build_odgather(nod)(h2, od0, od1)

    ws1a_p = jnp.pad(W_s1[:hd], ((0, _HP - hd), (0, 0)))
    ws1b_p = jnp.pad(W_s1[hd:], ((0, _HP - hd), (0, 0)))
    wk1h_p = jnp.pad(W_k1[:hd], ((0, _HP - hd), (0, 0)))
    scores_t, kc, ki = pl.pallas_call(
        functools.partial(_head_body, n),
        out_shape=[
            jax.ShapeDtypeStruct((1, nod), jnp.float32),
            jax.ShapeDtypeStruct((1, 1), jnp.float32),
            jax.ShapeDtypeStruct((1, 1), jnp.int32),
        ],
    )(o0, o1, ws1a_p, ws1b_p, b_s1.reshape(1, hd),
      W_s2.reshape(1, hd), b_s2.reshape(1, 1), msum,
      traffic_stats.reshape(1, 4), wk1h_p, W_k1[hd:],
      b_k1.reshape(1, 32), W_k2, b_k2.reshape(1, 16), W_k3,
      b_k3.reshape(1, 1))

    return (scores_t.reshape(nod), kc.reshape(()), ki.reshape(()))
